# baseline (speedup 1.0000x reference)
import functools

import jax
import jax.numpy as jnp
from jax import lax
from jax.experimental import pallas as pl
from jax.experimental.pallas import tpu as pltpu
from jax.experimental.pallas import tpu_sc as plsc

CHANNEL = 128
SIZE0, SIZE1 = 4, 4
EMB_DIM = CHANNEL * SIZE0 * SIZE1
BATCH = 16384
NC, NS = 2, 16
NW = NC * NS
B_PER_W = BATCH // NW
CHUNK = 16
NCHUNK = B_PER_W // CHUNK


_mesh = plsc.VectorSubcoreMesh(core_axis_name="c", subcore_axis_name="s")


@functools.partial(
    pl.kernel,
    out_type=jax.ShapeDtypeStruct((BATCH, EMB_DIM), jnp.float32),
    mesh=_mesh,
    scratch_types=[
        pltpu.VMEM((B_PER_W,), jnp.int32),
        pltpu.VMEM((CHUNK, EMB_DIM), jnp.float32),
        pltpu.SemaphoreType.DMA,
    ],
)
def _lookup(digits_hbm, table_hbm, out_hbm, idx_v, rows_v, sem):
    wid = lax.axis_index("s") * NC + lax.axis_index("c")
    base = wid * B_PER_W
    pltpu.sync_copy(digits_hbm.at[pl.ds(base, B_PER_W)], idx_v)

    def body(g, carry):
        off = g * CHUNK
        pltpu.async_copy(
            table_hbm.at[idx_v.at[pl.ds(off, CHUNK)]], rows_v, sem
        ).wait()
        pltpu.sync_copy(rows_v, out_hbm.at[pl.ds(base + off, CHUNK)])
        return carry

    lax.fori_loop(0, NCHUNK, body, 0)


def kernel(digits, table):
    out = _lookup(digits, table)
    return out.reshape(-1, CHANNEL, SIZE0, SIZE1)


# speedup vs baseline: 1.5844x; 1.5844x over previous
import functools

import jax
import jax.numpy as jnp
from jax import lax
from jax.experimental import pallas as pl
from jax.experimental.pallas import tpu as pltpu
from jax.experimental.pallas import tpu_sc as plsc

CHANNEL = 128
SIZE0, SIZE1 = 4, 4
EMB_DIM = CHANNEL * SIZE0 * SIZE1
BATCH = 16384
VOCAB = 10
NC, NS = 2, 16
NW = NC * NS
B_PER_W = BATCH // NW


_mesh = plsc.VectorSubcoreMesh(core_axis_name="c", subcore_axis_name="s")


@functools.partial(
    pl.kernel,
    out_type=jax.ShapeDtypeStruct((BATCH, EMB_DIM), jnp.float32),
    mesh=_mesh,
    scratch_types=[
        pltpu.VMEM((B_PER_W,), jnp.int32),
        pltpu.VMEM((VOCAB, EMB_DIM), jnp.float32),
        pltpu.SemaphoreType.DMA,
    ],
)
def _lookup(digits_hbm, table_hbm, out_hbm, idx_v, table_v, sem):
    wid = lax.axis_index("s") * NC + lax.axis_index("c")
    base = wid * B_PER_W
    pltpu.sync_copy(digits_hbm.at[pl.ds(base, B_PER_W)], idx_v)
    pltpu.sync_copy(table_hbm, table_v)

    def fire(g, carry):
        goff = g * 16
        vec = idx_v[pl.ds(goff, 16)]
        for k in range(16):
            row = vec[k]
            pltpu.async_copy(
                table_v.at[pl.ds(row, 1)],
                out_hbm.at[pl.ds(base + goff + k, 1)],
                sem,
            )
        return carry

    lax.fori_loop(0, B_PER_W // 16, fire, 0)

    def drain(j, carry):
        pltpu.make_async_copy(
            table_hbm.at[pl.ds(0, 1)], table_v.at[pl.ds(0, 1)], sem
        ).wait()
        return carry

    lax.fori_loop(0, B_PER_W, drain, 0)


def kernel(digits, table):
    out = _lookup(digits, table)
    return out.reshape(-1, CHANNEL, SIZE0, SIZE1)
